# baseline (speedup 1.0000x reference)
import functools

import jax
import jax.numpy as jnp
from jax import lax
from jax.experimental import pallas as pl
from jax.experimental.pallas import tpu as pltpu
from jax.experimental.pallas import tpu_sc as plsc

NC = 2
NS = 16
NW = NC * NS
CHUNK = 128
LANES = 16


def _sc_relayout(table_t, rem_lin, *, vocab, dim):
    vb = 256
    out_w = vb * dim
    nfull = vocab // vb
    rem = vocab - nfull * vb
    slots = -(-nfull // NW)
    assert slots % 2 == 1

    mesh = plsc.VectorSubcoreMesh(
        core_axis_name="c", subcore_axis_name="s", num_cores=NC,
        num_subcores=NS)

    @functools.partial(
        pl.kernel,
        out_type=jax.ShapeDtypeStruct((vocab * dim,), jnp.float32),
        mesh=mesh,
        compiler_params=pltpu.CompilerParams(needs_layout_passes=False),
        scratch_types=[
            pltpu.VMEM((dim, vb), jnp.float32),
            pltpu.VMEM((dim, vb), jnp.float32),
            pltpu.VMEM((out_w,), jnp.float32),
            pltpu.VMEM((out_w,), jnp.float32),
            pltpu.VMEM((max(rem, 1) * dim,), jnp.float32),
            pltpu.SemaphoreType.DMA,
            pltpu.SemaphoreType.DMA,
            pltpu.SemaphoreType.DMA,
            pltpu.SemaphoreType.DMA,
        ],
    )
    def body(tab_hbm, rem_hbm, lin_hbm, slab0_v, slab1_v, out0_v, out1_v,
             rem_v, si0, si1, so0, so1):
        w = lax.axis_index("s") * NC + lax.axis_index("c")
        slabs = (slab0_v, slab1_v)
        outs = (out0_v, out1_v)
        sin = (si0, si1)
        sout = (so0, so1)
        iota = lax.iota(jnp.int32, 16)

        def blk(j):
            return jnp.minimum(w + NW * j, nfull - 1)

        def start_in(j, t):
            pltpu.async_copy(
                tab_hbm.at[pl.ds(0, dim), pl.ds(blk(j) * vb, vb)],
                slabs[t], sin[t])

        def wait_in(t):
            pltpu.make_async_copy(
                tab_hbm.at[pl.ds(0, dim), pl.ds(0, vb)],
                slabs[t], sin[t]).wait()

        def start_out(j, t):
            pltpu.async_copy(outs[t],
                             lin_hbm.at[pl.ds(blk(j) * out_w, out_w)],
                             sout[t])

        def wait_out(t):
            pltpu.make_async_copy(outs[t],
                                  lin_hbm.at[pl.ds(0, out_w)],
                                  sout[t]).wait()

        def compute(t):
            slab = slabs[t]
            outr = outs[t]

            def vbody(v, _):
                cv = jnp.full((16,), v, jnp.int32)
                ov = jnp.full((16,), v * dim, jnp.int32) + iota
                for d0 in range(0, dim, 16):
                    x = plsc.load_gather(slab, [iota + d0, cv])
                    plsc.store_scatter(outr, [ov + d0], x)
                return 0

            lax.fori_loop(0, vb, vbody, 0)

        start_in(0, 0)
        start_in(1, 1)

        @pl.loop(0, slots - 1, step=2)
        def _(jj):
            for t in range(2):
                j = jj + t
                wait_in(t)

                @pl.when(j >= 2)
                def _():
                    wait_out(t)

                compute(t)
                start_out(j, t)

                @pl.when(j < slots - 2)
                def _():
                    start_in(j + 2, t)

        wait_in(0)
        wait_out(0)
        compute(0)
        start_out(slots - 1, 0)
        wait_out(1)
        wait_out(0)

        if rem:
            @pl.when(w == 0)
            def _():
                pltpu.sync_copy(rem_hbm, rem_v)
                pltpu.sync_copy(
                    rem_v, lin_hbm.at[pl.ds(nfull * out_w, rem * dim)])

    return body(table_t, rem_lin)


def _sc_gather(text2d, table, *, n_tok, batch, dim):
    head_chunks = batch // CHUNK
    kt = (n_tok - batch) // (NW * CHUNK)
    nchunks = 1 + kt

    mesh = plsc.VectorSubcoreMesh(
        core_axis_name="c", subcore_axis_name="s", num_cores=NC,
        num_subcores=NS)

    assert nchunks % 2 == 0

    @functools.partial(
        pl.kernel,
        out_type=[
            jax.ShapeDtypeStruct((batch, dim), jnp.float32),
            jax.ShapeDtypeStruct((NW, dim), jnp.float32),
        ],
        mesh=mesh,
        compiler_params=pltpu.CompilerParams(use_tc_tiling_on_sc=False),
        scratch_types=[
            pltpu.VMEM((nchunks * CHUNK,), jnp.int32),
            pltpu.VMEM((2, CHUNK, dim), jnp.float32),
            pltpu.VMEM((dim,), jnp.float32),
            pltpu.SemaphoreType.DMA,
            pltpu.SemaphoreType.DMA,
        ],
    )
    def body(text_hbm, table_hbm, rows_hbm, parts_hbm, idx_v, rows_v,
             stage_v, sem0, sem1):
        w = lax.axis_index("s") * NC + lax.axis_index("c")
        sems = (sem0, sem1)
        pltpu.sync_copy(text_hbm.at[pl.ds(w * CHUNK, CHUNK)],
                        idx_v.at[pl.ds(0, CHUNK)])
        pltpu.sync_copy(
            text_hbm.at[pl.ds((head_chunks + w * kt) * CHUNK, kt * CHUNK)],
            idx_v.at[pl.ds(CHUNK, kt * CHUNK)])

        def start(g, slot):
            pltpu.async_copy(
                table_hbm.at[idx_v.at[pl.ds(g * CHUNK, CHUNK)]],
                rows_v.at[slot], sems[slot])

        def wait(slot):
            pltpu.make_async_copy(
                table_hbm.at[idx_v.at[pl.ds(0, CHUNK)]],
                rows_v.at[slot], sems[slot]).wait()

        def consume(g, slot, accs):
            @pl.when(g == 0)
            def _():
                pltpu.sync_copy(rows_v.at[slot],
                                rows_hbm.at[pl.ds(w * CHUNK, CHUNK)])

            first = jnp.where(
                g == 0, jnp.where(w == NW - 1, CHUNK - 1, CHUNK), 0)

            def row_body(i, a):
                return tuple(
                    a[k] + rows_v[slot, i, pl.ds(LANES * k, LANES)]
                    for k in range(dim // LANES))

            return lax.fori_loop(first, CHUNK, row_body, accs)

        zero = jnp.zeros((LANES,), jnp.float32)
        start(0, 0)
        start(1, 1)

        @pl.loop(0, nchunks - 2, step=2,
                 init_carry=(zero,) * (dim // LANES))
        def accs_loop(j, accs):
            for t in range(2):
                wait(t)
                accs = consume(j + t, t, accs)
                start(j + t + 2, t)
            return accs

        accs = accs_loop
        for t in range(2):
            wait(t)
            accs = consume(nchunks - 2 + t, t, accs)

        for k in range(dim // LANES):
            stage_v[pl.ds(LANES * k, LANES)] = accs[k]
        pltpu.sync_copy(stage_v, parts_hbm.at[w])

    return body(text2d, table)


def _tc_finish(rows, parts, W2, b2, *, batch, tail_count):
    def body(rows_ref, parts_ref, w_ref, b_ref, out_ref):
        tail_mean = jnp.sum(parts_ref[...], axis=0) * (1.0 / tail_count)
        rid = lax.broadcasted_iota(jnp.int32, (batch, 1), 0)
        pooled = jnp.where(rid == batch - 1, tail_mean[None, :],
                           rows_ref[...])
        out_ref[...] = (
            jnp.dot(pooled, w_ref[...].T,
                    preferred_element_type=jnp.float32) + b_ref[...])

    return pl.pallas_call(
        body,
        out_shape=jax.ShapeDtypeStruct((batch, W2.shape[0]), jnp.float32),
    )(rows, parts, W2, b2)


def kernel(text, offsets, table, W, b):
    n_tok = text.shape[0]
    batch = offsets.shape[0]
    vocab, dim = table.shape
    assert batch % (NW * CHUNK) == 0 and (n_tok - batch) % (NW * CHUNK) == 0
    rem_start = (vocab // 256) * 256
    lin = _sc_relayout(table.T, table[rem_start:].reshape(-1),
                       vocab=vocab, dim=dim)
    rows, parts = _sc_gather(text, lin.reshape(vocab, dim), n_tok=n_tok,
                             batch=batch, dim=dim)
    out = _tc_finish(rows, parts, W, b.reshape(1, -1), batch=batch,
                     tail_count=n_tok - (batch - 1))
    return out
